# baseline (speedup 1.0000x reference)
import functools

import jax
import jax.numpy as jnp
from jax import lax
from jax.experimental import pallas as pl
from jax.experimental.pallas import tpu as pltpu
from jax.experimental.pallas import tpu_sc as plsc

_NUM_CLASS = 1000
_C, _H, _W = 3, 256, 128
_BATCH = 256
_D = _C * _H * _W
_K = 16
_DC = _D // _K
_NC, _NS = 2, 16
_NW = _NC * _NS
_ROWS_PER_W = _BATCH // _NW
_LANES = 16

_mesh = plsc.VectorSubcoreMesh(core_axis_name="c", subcore_axis_name="s")


@functools.partial(
    pl.kernel,
    mesh=_mesh,
    out_type=jax.ShapeDtypeStruct((_BATCH * _K, _DC), jnp.float32),
    scratch_types=[
        pltpu.VMEM((_LANES,), jnp.int32),
        pltpu.SemaphoreType.DMA,
    ],
)
def _gather_rows(tbl_hbm, lab_hbm, out_hbm, lab_v, sem):
    wid = lax.axis_index("s") * _NC + lax.axis_index("c")
    row_base = wid * _ROWS_PER_W * _K
    lab_base = wid * _ROWS_PER_W

    pltpu.sync_copy(lab_hbm.at[pl.ds(lab_base, _ROWS_PER_W)],
                    lab_v.at[pl.ds(0, _ROWS_PER_W)])
    iota = lax.iota(jnp.int32, _LANES)
    labs = lab_v[...]

    handles = []
    for t in range(_ROWS_PER_W):
        lab = labs[t]
        handles.append(pltpu.async_copy(
            tbl_hbm.at[pl.ds(lab * _K, _K)],
            out_hbm.at[pl.ds(row_base + t * _K, _K)],
            sem))
    for h in handles:
        h.wait()


def kernel(label, learnable_person_info):
    tbl = learnable_person_info.reshape(_NUM_CLASS * _K, _DC)
    out = _gather_rows(tbl, label)
    return out.reshape(_BATCH, _C, _H, _W)


# speedup vs baseline: 6.3146x; 6.3146x over previous
import functools

import jax
import jax.numpy as jnp
from jax import lax
from jax.experimental import pallas as pl
from jax.experimental.pallas import tpu as pltpu
from jax.experimental.pallas import tpu_sc as plsc

_NUM_CLASS = 1000
_C, _H, _W = 3, 256, 128
_BATCH = 256
_D = _C * _H * _W
_K = 16
_DC = _D // _K
_NC, _NS = 2, 16
_NW = _NC * _NS
_ROWS_PER_W = _BATCH // _NW
_NSTAGE = 8
_G = _ROWS_PER_W * _K // _NSTAGE
_LANES = 16

_mesh = plsc.VectorSubcoreMesh(core_axis_name="c", subcore_axis_name="s")


@functools.partial(
    pl.kernel,
    mesh=_mesh,
    out_type=jax.ShapeDtypeStruct((_BATCH * _K, _DC), jnp.float32),
    scratch_types=[
        pltpu.VMEM((_LANES,), jnp.int32),
        pltpu.VMEM((_NSTAGE, _DC), jnp.float32),
        pltpu.VMEM((_NSTAGE, _DC), jnp.float32),
        pltpu.SemaphoreType.DMA,
        pltpu.SemaphoreType.DMA,
        pltpu.SemaphoreType.DMA,
        pltpu.SemaphoreType.DMA,
    ],
)
def _gather_rows(tbl_hbm, lab_hbm, out_hbm, lab_v, buf0, buf1,
                 sg0, sg1, sw0, sw1):
    wid = lax.axis_index("s") * _NC + lax.axis_index("c")
    row_base = wid * _ROWS_PER_W * _K
    lab_base = wid * _ROWS_PER_W

    pltpu.sync_copy(lab_hbm.at[pl.ds(lab_base, _ROWS_PER_W)],
                    lab_v.at[pl.ds(0, _ROWS_PER_W)])
    labs = lab_v[...]
    lab_s = [labs[t] for t in range(_ROWS_PER_W)]

    bufs = (buf0, buf1)
    sgs = (sg0, sg1)
    sws = (sw0, sw1)

    def start_gather(g):
        src = lab_s[g // 2] * _K + (g % 2) * _NSTAGE
        return pltpu.async_copy(
            tbl_hbm.at[pl.ds(src, _NSTAGE)], bufs[g % 2], sgs[g % 2])

    def start_write(g):
        return pltpu.async_copy(
            bufs[g % 2],
            out_hbm.at[pl.ds(row_base + g * _NSTAGE, _NSTAGE)],
            sws[g % 2])

    hw = [None] * _G
    hg = [None] * _G
    hg[0] = start_gather(0)
    for g in range(_G):
        hg[g].wait()
        if g + 1 < _G:
            if g >= 1:
                hw[g - 1].wait()
            hg[g + 1] = start_gather(g + 1)
        hw[g] = start_write(g)
    hw[_G - 2].wait()
    hw[_G - 1].wait()


def kernel(label, learnable_person_info):
    tbl = learnable_person_info.reshape(_NUM_CLASS * _K, _DC)
    out = _gather_rows(tbl, label)
    return out.reshape(_BATCH, _C, _H, _W)


# speedup vs baseline: 38.3976x; 6.0808x over previous
import functools

import jax
import jax.numpy as jnp
from jax import lax
from jax.experimental import pallas as pl
from jax.experimental.pallas import tpu as pltpu
from jax.experimental.pallas import tpu_sc as plsc

_NUM_CLASS = 1000
_C, _H, _W = 3, 256, 128
_BATCH = 256
_NC, _NS = 2, 16
_NW = _NC * _NS
_ROWS_PER_W = _BATCH // _NW
_G = _ROWS_PER_W * _C
_LANES = 16

_mesh = plsc.VectorSubcoreMesh(core_axis_name="c", subcore_axis_name="s")


@functools.partial(
    pl.kernel,
    mesh=_mesh,
    out_type=jax.ShapeDtypeStruct((_BATCH, _C, _H, _W), jnp.float32),
    scratch_types=[
        pltpu.VMEM((_LANES,), jnp.int32),
        pltpu.VMEM((_H, _W), jnp.float32),
        pltpu.VMEM((_H, _W), jnp.float32),
        pltpu.SemaphoreType.DMA,
        pltpu.SemaphoreType.DMA,
        pltpu.SemaphoreType.DMA,
        pltpu.SemaphoreType.DMA,
    ],
)
def _gather_rows(tbl_hbm, lab_hbm, out_hbm, lab_v, buf0, buf1,
                 sg0, sg1, sw0, sw1):
    wid = lax.axis_index("s") * _NC + lax.axis_index("c")
    lab_base = wid * _ROWS_PER_W

    pltpu.sync_copy(lab_hbm.at[pl.ds(lab_base, _ROWS_PER_W)],
                    lab_v.at[pl.ds(0, _ROWS_PER_W)])
    labs = lab_v[...]
    lab_s = [labs[t] for t in range(_ROWS_PER_W)]

    bufs = (buf0, buf1)
    sgs = (sg0, sg1)
    sws = (sw0, sw1)

    def start_gather(g):
        return pltpu.async_copy(
            tbl_hbm.at[lab_s[g // _C], g % _C], bufs[g % 2], sgs[g % 2])

    def start_write(g):
        return pltpu.async_copy(
            bufs[g % 2],
            out_hbm.at[lab_base + g // _C, g % _C],
            sws[g % 2])

    hw = [None] * _G
    hg = [None] * _G
    hg[0] = start_gather(0)
    for g in range(_G):
        hg[g].wait()
        if g + 1 < _G:
            if g >= 1:
                hw[g - 1].wait()
            hg[g + 1] = start_gather(g + 1)
        hw[g] = start_write(g)
    hw[_G - 2].wait()
    hw[_G - 1].wait()


def kernel(label, learnable_person_info):
    return _gather_rows(learnable_person_info, label)


# speedup vs baseline: 39.6182x; 1.0318x over previous
import functools

import jax
import jax.numpy as jnp
from jax import lax
from jax.experimental import pallas as pl
from jax.experimental.pallas import tpu as pltpu
from jax.experimental.pallas import tpu_sc as plsc

_NUM_CLASS = 1000
_C, _H, _W = 3, 256, 128
_BATCH = 256
_NC, _NS = 2, 16
_NW = _NC * _NS
_ROWS_PER_W = _BATCH // _NW
_G = _ROWS_PER_W * _C
_LANES = 16

_mesh = plsc.VectorSubcoreMesh(core_axis_name="c", subcore_axis_name="s")


@functools.partial(
    pl.kernel,
    mesh=_mesh,
    out_type=jax.ShapeDtypeStruct((_BATCH, _C, _H, _W), jnp.float32),
    scratch_types=[
        pltpu.VMEM((_LANES,), jnp.int32),
        pltpu.VMEM((_H, _W), jnp.float32),
        pltpu.VMEM((_H, _W), jnp.float32),
        pltpu.VMEM((_H, _W), jnp.float32),
        pltpu.SemaphoreType.DMA,
        pltpu.SemaphoreType.DMA,
        pltpu.SemaphoreType.DMA,
        pltpu.SemaphoreType.DMA,
        pltpu.SemaphoreType.DMA,
        pltpu.SemaphoreType.DMA,
    ],
)
def _gather_rows(tbl_hbm, lab_hbm, out_hbm, lab_v, buf0, buf1, buf2,
                 sg0, sg1, sg2, sw0, sw1, sw2):
    wid = lax.axis_index("s") * _NC + lax.axis_index("c")
    lab_base = wid * _ROWS_PER_W

    pltpu.sync_copy(lab_hbm.at[pl.ds(lab_base, _ROWS_PER_W)],
                    lab_v.at[pl.ds(0, _ROWS_PER_W)])
    labs = lab_v[...]
    lab_s = [labs[t] for t in range(_ROWS_PER_W)]

    bufs = (buf0, buf1, buf2)
    sgs = (sg0, sg1, sg2)
    sws = (sw0, sw1, sw2)
    _NB = 3

    def start_gather(g):
        return pltpu.async_copy(
            tbl_hbm.at[lab_s[g // _C], g % _C], bufs[g % _NB], sgs[g % _NB])

    def start_write(g):
        return pltpu.async_copy(
            bufs[g % _NB],
            out_hbm.at[lab_base + g // _C, g % _C],
            sws[g % _NB])

    hw = [None] * _G
    hg = [None] * _G
    hg[0] = start_gather(0)
    hg[1] = start_gather(1)
    for g in range(_G):
        hg[g].wait()
        if g + 2 < _G:
            if g >= 1:
                hw[g - 1].wait()
            hg[g + 2] = start_gather(g + 2)
        hw[g] = start_write(g)
    hw[_G - 3].wait()
    hw[_G - 2].wait()
    hw[_G - 1].wait()


def kernel(label, learnable_person_info):
    return _gather_rows(learnable_person_info, label)
